# Initial kernel scaffold; baseline (speedup 1.0000x reference)
import jax
import jax.numpy as jnp
from jax.experimental import pallas as pl


def kernel(feat, edge_index, fcW, fcb, qW, qb, kW, kb, vW, vb, n1s, n1b, ffW1, ffb1, ffW2, ffb2, n2s, n2b, ascale, abias, rscale, pW1, pb1, pW2, pb2, dW1, db1, dW2, db2):
    raise NotImplementedError("write your pallas kernel here")


# speedup vs baseline: 26.0213x; 26.0213x over previous
import functools

import jax
import jax.numpy as jnp
from jax import lax
from jax.experimental import pallas as pl
from jax.experimental.pallas import tpu as pltpu
from jax.experimental.pallas import tpu_sc as plsc

N = 10000
E = 160000
D = 384
NP = 10240
BLK = 256
GRID = NP // BLK
NW = 32
EP = 160256
EC = EP // NW
NB = NP // NW
CH = 64
NCH = NB // CH
H8 = 8
HD8 = D // H8

_mesh = plsc.VectorSubcoreMesh(core_axis_name="c", subcore_axis_name="s")


def _wid():
    return lax.axis_index("s") * 2 + lax.axis_index("c")


@functools.partial(
    pl.kernel,
    out_type=(jax.ShapeDtypeStruct((NW * NP,), jnp.int32),
              jax.ShapeDtypeStruct((NW * NP,), jnp.int32)),
    mesh=_mesh,
    compiler_params=pltpu.CompilerParams(needs_layout_passes=False),
    scratch_types=[pltpu.VMEM((EC,), jnp.int32),
                   pltpu.VMEM((EC,), jnp.int32),
                   pltpu.VMEM((NP,), jnp.int32),
                   pltpu.VMEM((NP,), jnp.int32)],
)
def _win_tables(src_hbm, dst_hbm, win_hbm, sel_hbm, src_v, dst_v, win_v, sel_v):
    wid = _wid()
    base = wid * EC
    pltpu.sync_copy(src_hbm.at[pl.ds(base, EC)], src_v)
    pltpu.sync_copy(dst_hbm.at[pl.ds(base, EC)], dst_v)
    neg = jnp.full((16,), -1, jnp.int32)

    zero = jnp.zeros((16,), jnp.int32)

    def init(i, c):
        win_v[pl.ds(i * 16, 16)] = neg
        sel_v[pl.ds(i * 16, 16)] = zero
        return c

    lax.fori_loop(0, NP // 16, init, 0)
    lane = lax.iota(jnp.int32, 16)

    def step(i, c):
        sl = pl.ds(i * 16, 16)
        d = dst_v[sl]
        s = src_v[sl]
        e = (base + i * 16) + lane
        keep = lane >= 0
        for j in range(1, 16):
            idx = lane + j
            dj = d.at[jnp.minimum(idx, 15)].get(mode='promise_in_bounds')
            keep = keep & ((dj != d) | (idx > 15))
        plsc.store_scatter(win_v, [d], e, mask=keep)
        plsc.store_scatter(sel_v, [d], s, mask=keep)
        return c

    lax.fori_loop(0, EC // 16, step, 0)
    pltpu.sync_copy(win_v, win_hbm.at[pl.ds(wid * NP, NP)])
    pltpu.sync_copy(sel_v, sel_hbm.at[pl.ds(wid * NP, NP)])


def _merge_body(win_ref, sel_ref, osel_ref, omask_ref):
    win = win_ref[...]
    sel = sel_ref[...]
    bw = jnp.max(win, axis=0, keepdims=True)
    eq = win == bw
    bs = jnp.sum(jnp.where(eq, sel, 0), axis=0, keepdims=True)
    osel_ref[...] = bs
    omask_ref[...] = (bw >= 0).astype(jnp.float32)


def _tc_merge(win_t, sel_t):
    full = pl.BlockSpec((NW, NP), lambda: (0, 0))
    out = pl.BlockSpec((1, NP), lambda: (0, 0))
    return pl.pallas_call(
        _merge_body,
        in_specs=[full, full],
        out_specs=[out, out],
        out_shape=[jax.ShapeDtypeStruct((1, NP), jnp.int32),
                   jax.ShapeDtypeStruct((1, NP), jnp.float32)],
    )(win_t.reshape(NW, NP), sel_t.reshape(NW, NP))


@functools.partial(
    pl.kernel,
    out_type=jax.ShapeDtypeStruct((NP, D), jnp.float32),
    mesh=_mesh,
    compiler_params=pltpu.CompilerParams(needs_layout_passes=False),
    scratch_types=[pltpu.VMEM((NB,), jnp.int32),
                   pltpu.VMEM((NB, D), jnp.float32),
                   pltpu.SemaphoreType.DMA,
                   pltpu.SemaphoreType.DMA],
)
def _sc_gather(table_hbm, idx_hbm, out_hbm, idx_v, rows_v, sem, sem2):
    wid = _wid()
    base = wid * NB
    pltpu.sync_copy(idx_hbm.at[pl.ds(base, NB)], idx_v)
    copies = [
        pltpu.async_copy(table_hbm.at[idx_v.at[pl.ds(c * CH, CH)]],
                         rows_v.at[pl.ds(c * CH, CH)], sem)
        for c in range(NCH)
    ]
    outs = []
    for c, cp in enumerate(copies):
        cp.wait()
        outs.append(pltpu.async_copy(rows_v.at[pl.ds(c * CH, CH)],
                                     out_hbm.at[pl.ds(base + c * CH, CH)],
                                     sem2))
    for op in outs:
        op.wait()


def _ln(x, s, b):
    mu = jnp.mean(x, axis=1, keepdims=True)
    xc = x - mu
    var = jnp.mean(xc * xc, axis=1, keepdims=True)
    return xc * lax.rsqrt(var + 1e-5) * s + b


def _dot(a, b):
    return jnp.dot(a, b, preferred_element_type=jnp.float32)


def _hf_body(h_ref, fcW_ref, fcb_ref, hf_ref):
    hf_ref[...] = _dot(h_ref[...], fcW_ref[...]) + fcb_ref[...]


def _head_masks():
    dd = lax.broadcasted_iota(jnp.int32, (D, H8), 0) // HD8
    hh = lax.broadcasted_iota(jnp.int32, (D, H8), 1)
    seg = (dd == hh).astype(jnp.float32)
    dd2 = lax.broadcasted_iota(jnp.int32, (H8, D), 1) // HD8
    hh2 = lax.broadcasted_iota(jnp.int32, (H8, D), 0)
    seg_t = (dd2 == hh2).astype(jnp.float32)
    return seg, seg_t


def _roll(x, j):
    if j == 0:
        return x
    return jnp.concatenate([x[:, HD8 * j:], x[:, :HD8 * j]], axis=1)


def _attn_ffn_body(hf_ref, hfs_ref, mask_ref, scal_ref,
                   qW_ref, qb_ref, kW_ref, kb_ref, vW_ref, vb_ref,
                   n1s_ref, n1b_ref, ffW1_ref, ffb1_ref, ffW2_ref, ffb2_ref,
                   n2s_ref, n2b_ref, fcWn_ref, fcbn_ref, out_ref):
    hf = hf_ref[...]
    hfs = hfs_ref[...]
    q = _dot(hf, qW_ref[...]) + qb_ref[...]
    k = _dot(hfs, kW_ref[...]) + kb_ref[...]
    v = _dot(hfs, vW_ref[...]) + vb_ref[...]
    ascale = scal_ref[0, 0]
    abias = scal_ref[0, 1]
    rscale = scal_ref[0, 2]
    seg, seg_t = _head_masks()
    sc = ascale / (HD8 ** 0.5)
    logits = []
    vs = []
    for j in range(H8):
        kj = _roll(k, j)
        vs.append(_roll(v, j))
        logits.append(_dot(q * kj, seg) * sc + abias)
    m = logits[0]
    for j in range(1, H8):
        m = jnp.maximum(m, logits[j])
    es = [jnp.exp(l - m) for l in logits]
    tot = es[0]
    for j in range(1, H8):
        tot = tot + es[j]
    inv = 1.0 / tot
    he = jnp.zeros_like(hf)
    for j in range(H8):
        he = he + _dot(es[j] * inv, seg_t) * vs[j]
    msk = mask_ref[...]
    h_upd = hf + msk * (he - hf)
    h1 = _ln(hf + h_upd * rscale, n1s_ref[...], n1b_ref[...])
    ffn = jnp.maximum(_dot(h1, ffW1_ref[...]) + ffb1_ref[...], 0.0)
    ffn = _dot(ffn, ffW2_ref[...]) + ffb2_ref[...]
    h2 = _ln(h1 + ffn * rscale, n2s_ref[...], n2b_ref[...])
    out_ref[...] = _dot(h2, fcWn_ref[...]) + fcbn_ref[...]


def _l3_final_body(hf_ref, hfs_ref, mask_ref, scal_ref, vW_ref, vb_ref,
                   n1s_ref, n1b_ref, ffW1_ref, ffb1_ref, ffW2_ref, ffb2_ref,
                   n2s_ref, n2b_ref, pW1_ref, pb1_ref, pW2_ref, pb2_ref,
                   dW1_ref, db1_ref, dW2_ref, db2_ref, pitch_ref, dur_ref):
    hf = hf_ref[...]
    vsel = _dot(hfs_ref[...], vW_ref[...]) + vb_ref[...]
    rscale = scal_ref[0, 0]
    msk = mask_ref[...]
    h_upd = hf + msk * (vsel - hf)
    h1 = _ln(hf + h_upd * rscale, n1s_ref[...], n1b_ref[...])
    ffn = jnp.maximum(_dot(h1, ffW1_ref[...]) + ffb1_ref[...], 0.0)
    ffn = _dot(ffn, ffW2_ref[...]) + ffb2_ref[...]
    h = _ln(h1 + ffn * rscale, n2s_ref[...], n2b_ref[...])
    p = jnp.maximum(_dot(h, pW1_ref[...]) + pb1_ref[...], 0.0)
    pitch_ref[...] = _dot(p, pW2_ref[...]) + pb2_ref[...]
    dd = jnp.maximum(_dot(h, dW1_ref[...]) + db1_ref[...], 0.0)
    dur_ref[...] = _dot(dd, dW2_ref[...]) + db2_ref[...]


def _row_spec(width=D):
    return pl.BlockSpec((BLK, width), lambda i: (i, 0))


def _full_spec(shape):
    nd = len(shape)
    return pl.BlockSpec(shape, lambda i: (0,) * nd)


_SMEM_SPEC = pl.BlockSpec(memory_space=pltpu.SMEM)


def _tc_hf(h, fcW, fcb):
    return pl.pallas_call(
        _hf_body,
        grid=(GRID,),
        in_specs=[_row_spec(), _full_spec((D, D)), _full_spec((1, D))],
        out_specs=_row_spec(),
        out_shape=jax.ShapeDtypeStruct((NP, D), jnp.float32),
    )(h, fcW, fcb.reshape(1, D))


def _tc_attn_ffn(hf, hfs, mask2, scal, qW, qb, kW, kb, vW, vb,
                 n1s, n1b, ffW1, ffb1, ffW2, ffb2, n2s, n2b, fcWn, fcbn):
    w = _full_spec((D, D))
    b = _full_spec((1, D))
    r = _row_spec()
    mspec = pl.BlockSpec((BLK, 1), lambda i: (i, 0))
    return pl.pallas_call(
        _attn_ffn_body,
        grid=(GRID,),
        in_specs=[r, r, mspec, _SMEM_SPEC, w, b, w, b, w, b,
                  b, b, w, b, w, b, b, b, w, b],
        out_specs=r,
        out_shape=jax.ShapeDtypeStruct((NP, D), jnp.float32),
    )(hf, hfs, mask2, scal, qW, qb.reshape(1, D), kW, kb.reshape(1, D),
      vW, vb.reshape(1, D), n1s.reshape(1, D), n1b.reshape(1, D),
      ffW1, ffb1.reshape(1, D), ffW2, ffb2.reshape(1, D),
      n2s.reshape(1, D), n2b.reshape(1, D), fcWn, fcbn.reshape(1, D))


def _tc_l3_final(hf, hfs, mask2, scal, vW, vb, n1s, n1b, ffW1, ffb1,
                 ffW2, ffb2, n2s, n2b, pW1, pb1, pW2, pb2,
                 dW1, db1, dW2, db2):
    w = _full_spec((D, D))
    b = _full_spec((1, D))
    r = _row_spec()
    mspec = pl.BlockSpec((BLK, 1), lambda i: (i, 0))
    return pl.pallas_call(
        _l3_final_body,
        grid=(GRID,),
        in_specs=[r, r, mspec, _SMEM_SPEC, w, b, b, b, w, b, w, b, b, b,
                  w, b, _full_spec((D, 88)), _full_spec((1, 88)),
                  w, b, _full_spec((D, 32)), _full_spec((1, 32))],
        out_specs=[pl.BlockSpec((BLK, 88), lambda i: (i, 0)),
                   pl.BlockSpec((BLK, 32), lambda i: (i, 0))],
        out_shape=[jax.ShapeDtypeStruct((NP, 88), jnp.float32),
                   jax.ShapeDtypeStruct((NP, 32), jnp.float32)],
    )(hf, hfs, mask2, scal, vW, vb.reshape(1, D),
      n1s.reshape(1, D), n1b.reshape(1, D),
      ffW1, ffb1.reshape(1, D), ffW2, ffb2.reshape(1, D),
      n2s.reshape(1, D), n2b.reshape(1, D),
      pW1, pb1.reshape(1, D), pW2, pb2.reshape(1, 88),
      dW1, db1.reshape(1, D), dW2, db2.reshape(1, 32))


def kernel(feat, edge_index, fcW, fcb, qW, qb, kW, kb, vW, vb, n1s, n1b,
           ffW1, ffb1, ffW2, ffb2, n2s, n2b, ascale, abias, rscale,
           pW1, pb1, pW2, pb2, dW1, db1, dW2, db2):
    src = jnp.concatenate([edge_index[0], jnp.zeros((EP - E,), jnp.int32)])
    dst = jnp.concatenate([edge_index[1], jnp.full((EP - E,), N, jnp.int32)])
    win_t, sel_t = _win_tables(src, dst)
    sel2d, mask2d = _tc_merge(win_t, sel_t)
    sel = sel2d.reshape(NP)
    mask2 = mask2d.reshape(NP, 1)

    h0 = jnp.zeros((NP, D), jnp.float32).at[:N].set(feat)
    hf = _tc_hf(h0, fcW[0], fcb[0])
    for i in range(2):
        hfs = _sc_gather(hf, sel)
        scal = jnp.stack([ascale[i], abias[i], rscale[i]]).reshape(1, 3)
        hf = _tc_attn_ffn(hf, hfs, mask2, scal, qW[i], qb[i], kW[i], kb[i],
                          vW[i], vb[i], n1s[i], n1b[i], ffW1[i], ffb1[i],
                          ffW2[i], ffb2[i], n2s[i], n2b[i],
                          fcW[i + 1], fcb[i + 1])
    hfs = _sc_gather(hf, sel)
    scal3 = rscale[2].reshape(1, 1)
    pitch, dur = _tc_l3_final(hf, hfs, mask2, scal3, vW[2], vb[2],
                              n1s[2], n1b[2], ffW1[2], ffb1[2], ffW2[2],
                              ffb2[2], n2s[2], n2b[2], pW1, pb1, pW2, pb2,
                              dW1, db1, dW2, db2)
    return (pitch[:N], dur[:N])
